# baseline (speedup 1.0000x reference)
import functools

import jax
import jax.numpy as jnp
import numpy as np
from jax.experimental import pallas as pl
from jax.experimental.pallas import tpu as pltpu

N = 10000
E = 160000
H = 64
L = 2
T = 4
G = 128
AVG_LOG = float(np.mean(np.log(np.arange(10, dtype=np.float64) + 1.0)))

BN = 1000
CE = 2000
NBLK = N // BN
EBLK = E // CE
SM = pltpu.MemorySpace.SMEM
_SEQ = pltpu.CompilerParams(dimension_semantics=("arbitrary",))


def _enc_kernel(x_ref, nw_ref, nb_ref, wi_ref, wj_ref,
                ew8_ref, eb_ref, eew1_ref, eeb1_ref, we1_ref, pb1_ref,
                eew2_ref, eeb2_ref, we2_ref, pb2_ref,
                h_ref, a_ref, b_ref, te1_ref, te2_ref):
    h = jnp.dot(x_ref[...], nw_ref[...], preferred_element_type=jnp.float32)
    h = h + nb_ref[...]
    h_ref[...] = h
    a_ref[...] = jnp.dot(h, wi_ref[...], preferred_element_type=jnp.float32)
    b_ref[...] = jnp.dot(h, wj_ref[...], preferred_element_type=jnp.float32)

    @pl.when(pl.program_id(0) == 0)
    def _():
        ea = ew8_ref[...] + eb_ref[...]
        t1 = jnp.dot(ea, eew1_ref[...], preferred_element_type=jnp.float32) + eeb1_ref[...]
        te1_ref[...] = jnp.dot(t1, we1_ref[...], preferred_element_type=jnp.float32) + pb1_ref[...]
        t2 = jnp.dot(ea, eew2_ref[...], preferred_element_type=jnp.float32) + eeb2_ref[...]
        te2_ref[...] = jnp.dot(t2, we2_ref[...], preferred_element_type=jnp.float32) + pb2_ref[...]


def _stats_kernel(src_ref, dst_ref, attr_ref, b_ref, te_ref,
                  ss_ref, me_ref, deg_ref, *, with_deg):
    @pl.when(pl.program_id(0) == 0)
    def _():
        ss_ref[...] = jnp.zeros_like(ss_ref)
        me_ref[...] = jnp.full_like(me_ref, jnp.inf)
        if with_deg:
            deg_ref[...] = jnp.zeros_like(deg_ref)

    def body(i, carry):
        s = src_ref[0, 0, i]
        d = dst_ref[0, 0, i]
        a = attr_ref[0, 0, i]
        c = b_ref[pl.ds(s, 1), :] + te_ref[pl.ds(a, 1), :]
        ss_ref[pl.ds(d, 1), :] += jnp.concatenate([c, c * c], axis=1)
        me_ref[pl.ds(d, 1), :] = jnp.minimum(
            me_ref[pl.ds(d, 1), :], jnp.concatenate([c, -c], axis=1))
        if with_deg:
            deg_ref[pl.ds(d, 1), :] += 1.0
        return carry

    jax.lax.fori_loop(0, CE, body, 0)


def _run_stats(src3, dst3, attr3, b_tab, te, with_deg):
    full = lambda shp: pl.BlockSpec(shp, lambda i: (0,) * len(shp))
    idx_spec = pl.BlockSpec((1, 1, CE), lambda i: (i, 0, 0), memory_space=SM)
    C2 = 2 * T * H
    out_shapes = [jax.ShapeDtypeStruct((N, C2), jnp.float32)] * 2
    out_specs = [full((N, C2))] * 2
    if with_deg:
        out_shapes = out_shapes + [jax.ShapeDtypeStruct((N, 1), jnp.float32)]
        out_specs = out_specs + [full((N, 1))]
        kern = functools.partial(_stats_kernel, with_deg=True)
    else:
        def kern(src_ref, dst_ref, attr_ref, b_ref, te_ref, ss_ref, me_ref):
            _stats_kernel(src_ref, dst_ref, attr_ref, b_ref, te_ref,
                          ss_ref, me_ref, None, with_deg=False)
    return pl.pallas_call(
        kern,
        grid=(EBLK,),
        in_specs=[idx_spec, idx_spec, idx_spec, full((N, T * H)), full((8, T * H))],
        out_specs=out_specs,
        out_shape=out_shapes,
        compiler_params=_SEQ,
    )(src3, dst3, attr3, b_tab, te)


def _node_kernel(h_ref, a_ref, ss_ref, me_ref, deg_ref,
                 px_ref, pa_ref, pb_ref, pc_ref, postb_ref, lw_ref, lb_ref,
                 y_ref, ysum_ref, ysq_ref):
    d = deg_ref[...]
    pos = d > 0.0
    dc = jnp.maximum(d, 1.0)
    inv = 1.0 / dc
    a = a_ref[...]
    C = T * H
    s1 = ss_ref[:, :C] * inv
    s2 = ss_ref[:, C:] * inv
    mean = jnp.where(pos, a + s1, 0.0)
    std = jnp.sqrt(jnp.maximum(s2 - s1 * s1, 0.0) + 1e-5)
    mn = jnp.where(pos, a + me_ref[:, :C], 0.0)
    mx = jnp.where(pos, a - me_ref[:, C:], 0.0)
    agg = jnp.concatenate([mean, mn, mx, std], axis=1)
    dl = jnp.log(dc + 1.0)
    sc1 = dl * (1.0 / AVG_LOG)
    sc2 = AVG_LOG / dl
    h = h_ref[...]
    outs = jnp.dot(h, px_ref[...], preferred_element_type=jnp.float32)
    outs += jnp.dot(agg, pa_ref[...], preferred_element_type=jnp.float32)
    outs += sc1 * jnp.dot(agg, pb_ref[...], preferred_element_type=jnp.float32)
    outs += sc2 * jnp.dot(agg, pc_ref[...], preferred_element_type=jnp.float32)
    outs += postb_ref[...]
    y = jnp.dot(outs, lw_ref[...], preferred_element_type=jnp.float32) + lb_ref[...]
    y_ref[...] = y

    @pl.when(pl.program_id(0) == 0)
    def _():
        ysum_ref[...] = jnp.zeros_like(ysum_ref)
        ysq_ref[...] = jnp.zeros_like(ysq_ref)

    ysum_ref[...] += jnp.sum(y, axis=0, keepdims=True)
    ysq_ref[...] += jnp.sum(y * y, axis=0, keepdims=True)


def _run_node(h, a_tab, ss, me, deg, px, pa, pb, pc, postb, lw, lb):
    blk = lambda w: pl.BlockSpec((BN, w), lambda i: (i, 0))
    full = lambda shp: pl.BlockSpec(shp, lambda i: (0,) * len(shp))
    C = T * H
    return pl.pallas_call(
        _node_kernel,
        grid=(NBLK,),
        in_specs=[blk(H), blk(C), blk(2 * C), blk(2 * C), blk(1),
                  full((H, H)), full((4 * C, H)), full((4 * C, H)),
                  full((4 * C, H)), full((1, H)), full((H, H)), full((1, H))],
        out_specs=[blk(H), full((1, H)), full((1, H))],
        out_shape=[jax.ShapeDtypeStruct((N, H), jnp.float32),
                   jax.ShapeDtypeStruct((1, H), jnp.float32),
                   jax.ShapeDtypeStruct((1, H), jnp.float32)],
        compiler_params=_SEQ,
    )(h, a_tab, ss, me, deg, px, pa, pb, pc, postb, lw, lb)


def _mid_kernel(y_ref, ysum_ref, ysq_ref, g_ref, bb_ref, h0_ref,
                wi_ref, wj_ref, h1_ref, a_ref, b_ref):
    m = ysum_ref[...] * (1.0 / N)
    v = ysq_ref[...] * (1.0 / N) - m * m
    inv = jax.lax.rsqrt(v + 1e-5)
    h1 = jnp.maximum((y_ref[...] - m) * inv * g_ref[...] + bb_ref[...], 0.0)
    h1 = h1 + h0_ref[...]
    h1_ref[...] = h1
    a_ref[...] = jnp.dot(h1, wi_ref[...], preferred_element_type=jnp.float32)
    b_ref[...] = jnp.dot(h1, wj_ref[...], preferred_element_type=jnp.float32)


def _head_kernel(y_ref, ysum_ref, ysq_ref, g_ref, bb_ref, h1_ref, batch_ref,
                 yt_ref, l1w_ref, l1b_ref, l3w_ref, l3b_ref,
                 xcat_ref, loss_ref):
    m = ysum_ref[...] * (1.0 / N)
    v = ysq_ref[...] * (1.0 / N) - m * m
    inv = jax.lax.rsqrt(v + 1e-5)
    h2 = jnp.maximum((y_ref[...] - m) * inv * g_ref[...] + bb_ref[...], 0.0)
    h2 = h2 + h1_ref[...]
    oh = (batch_ref[...] ==
          jax.lax.broadcasted_iota(jnp.int32, (BN, G), 1)).astype(jnp.float32)

    @pl.when(pl.program_id(0) == 0)
    def _():
        xcat_ref[...] = jnp.zeros_like(xcat_ref)

    xcat_ref[...] += jax.lax.dot_general(
        oh, h2, (((0,), (0,)), ((), ())), preferred_element_type=jnp.float32)

    @pl.when(pl.program_id(0) == NBLK - 1)
    def _():
        xc = jnp.maximum(
            jnp.dot(xcat_ref[...], l1w_ref[...], preferred_element_type=jnp.float32)
            + l1b_ref[...], 0.0)
        p = jnp.dot(xc, l3w_ref[...], preferred_element_type=jnp.float32) + l3b_ref[...]
        dd = jnp.abs(p - yt_ref[...])
        lv = jnp.where(dd < 0.5, dd * dd, dd - 0.25)
        loss_ref[...] = jnp.sum(lv, keepdims=True).reshape(1, 1) * (1.0 / G)


def kernel(x, edge_index, edge_attr, batch, y, node_W, node_b, eenc_W, eenc_b,
           pna_ee_W, pna_ee_b, pre_W, pre_b, post_W, post_b, lin_W, lin_b,
           bn_g, bn_b, lin1_W, lin1_b, lin3_W, lin3_b):
    f32 = jnp.float32
    C = T * H
    src3 = edge_index[0].reshape(EBLK, 1, CE)
    dst3 = edge_index[1].reshape(EBLK, 1, CE)
    attr3 = edge_attr.reshape(EBLK, 1, CE)
    batch2 = batch.reshape(N, 1)
    yt = y.reshape(G, 1)

    wi, wj, we, pbv, px, pabc, postb = [], [], [], [], [], [], []
    eye4 = jnp.eye(T, dtype=f32)
    for l in range(L):
        pw = pre_W[l]
        wi.append(pw[:, :H, :].transpose(1, 0, 2).reshape(H, C))
        wj.append(pw[:, H:2 * H, :].transpose(1, 0, 2).reshape(H, C))
        we.append(pw[:, 2 * H:, :].transpose(1, 0, 2).reshape(H, C))
        pbv.append(pre_b[l].reshape(1, C))
        po = post_W[l]
        px.append(po[:, :H, :].transpose(1, 0, 2).reshape(H, H))
        pw2 = po[:, H:, :].reshape(T, 12, H, H // T)
        groups = []
        for g in range(3):
            bg = pw2[:, g * 4:(g + 1) * 4].transpose(1, 0, 2, 3)
            z = jnp.einsum('stoj,tu->stouj', bg, eye4)
            groups.append(z.reshape(4 * C, H))
        pabc.append(groups)
        postb.append(post_b[l].reshape(1, H))
    eenc_W8 = jnp.pad(eenc_W, ((0, 4), (0, 0)))
    eenc_b2 = eenc_b.reshape(1, H)
    te_pb = []
    for l in range(L):
        te_pb.append((pna_ee_W[l], pna_ee_b[l].reshape(1, H), we[l], pbv[l]))

    blk = lambda w: pl.BlockSpec((BN, w), lambda i: (i, 0))
    full = lambda shp: pl.BlockSpec(shp, lambda i: (0,) * len(shp))
    h0, a1, b1, te1, te2 = pl.pallas_call(
        _enc_kernel,
        grid=(NBLK,),
        in_specs=[blk(9), full((9, H)), full((1, H)), full((H, C)), full((H, C)),
                  full((8, H)), full((1, H)),
                  full((H, H)), full((1, H)), full((H, C)), full((1, C)),
                  full((H, H)), full((1, H)), full((H, C)), full((1, C))],
        out_specs=[blk(H), blk(C), blk(C), full((8, C)), full((8, C))],
        out_shape=[jax.ShapeDtypeStruct((N, H), f32),
                   jax.ShapeDtypeStruct((N, C), f32),
                   jax.ShapeDtypeStruct((N, C), f32),
                   jax.ShapeDtypeStruct((8, C), f32),
                   jax.ShapeDtypeStruct((8, C), f32)],
        compiler_params=_SEQ,
    )(x, node_W, node_b.reshape(1, H), wi[0], wj[0],
      eenc_W8, eenc_b2, te_pb[0][0], te_pb[0][1], te_pb[0][2], te_pb[0][3],
      te_pb[1][0], te_pb[1][1], te_pb[1][2], te_pb[1][3])

    ss1, me1, deg = _run_stats(src3, dst3, attr3, b1, te1, with_deg=True)
    y1, ysum1, ysq1 = _run_node(h0, a1, ss1, me1, deg,
                                px[0], pabc[0][0], pabc[0][1], pabc[0][2],
                                postb[0], lin_W[0], lin_b[0].reshape(1, H))

    h1, a2, b2 = pl.pallas_call(
        _mid_kernel,
        grid=(NBLK,),
        in_specs=[blk(H), full((1, H)), full((1, H)), full((1, H)), full((1, H)),
                  blk(H), full((H, C)), full((H, C))],
        out_specs=[blk(H), blk(C), blk(C)],
        out_shape=[jax.ShapeDtypeStruct((N, H), f32),
                   jax.ShapeDtypeStruct((N, C), f32),
                   jax.ShapeDtypeStruct((N, C), f32)],
        compiler_params=_SEQ,
    )(y1, ysum1, ysq1, bn_g[0].reshape(1, H), bn_b[0].reshape(1, H),
      h0, wi[1], wj[1])

    ss2, me2 = _run_stats(src3, dst3, attr3, b2, te2, with_deg=False)
    y2, ysum2, ysq2 = _run_node(h1, a2, ss2, me2, deg,
                                px[1], pabc[1][0], pabc[1][1], pabc[1][2],
                                postb[1], lin_W[1], lin_b[1].reshape(1, H))

    _, loss2d = pl.pallas_call(
        _head_kernel,
        grid=(NBLK,),
        in_specs=[blk(H), full((1, H)), full((1, H)), full((1, H)), full((1, H)),
                  blk(H), pl.BlockSpec((BN, 1), lambda i: (i, 0)),
                  full((G, 1)), full((H, H)), full((1, H)), full((H, 1)),
                  full((1, 1))],
        out_specs=[full((G, H)), full((1, 1))],
        out_shape=[jax.ShapeDtypeStruct((G, H), f32),
                   jax.ShapeDtypeStruct((1, 1), f32)],
        compiler_params=_SEQ,
    )(y2, ysum2, ysq2, bn_g[1].reshape(1, H), bn_b[1].reshape(1, H),
      h1, batch2, yt, lin1_W, lin1_b.reshape(1, H), lin3_W,
      lin3_b.reshape(1, 1))

    loss = loss2d.reshape(())
    return (loss, loss)


# speedup vs baseline: 1.3161x; 1.3161x over previous
import functools

import jax
import jax.numpy as jnp
import numpy as np
from jax.experimental import pallas as pl
from jax.experimental.pallas import tpu as pltpu

N = 10000
E = 160000
H = 64
L = 2
T = 4
G = 128
AVG_LOG = float(np.mean(np.log(np.arange(10, dtype=np.float64) + 1.0)))

BN = 1000
CE = 2000
NBLK = N // BN
EBLK = E // CE
SM = pltpu.MemorySpace.SMEM
_SEQ = pltpu.CompilerParams(dimension_semantics=("arbitrary",))


def _enc_kernel(x_ref, nw_ref, nb_ref, wi_ref, wj_ref,
                ew8_ref, eb_ref, eew1_ref, eeb1_ref, we1_ref, pb1_ref,
                eew2_ref, eeb2_ref, we2_ref, pb2_ref,
                h_ref, a_ref, b_ref, te1_ref, te2_ref):
    h = jnp.dot(x_ref[...], nw_ref[...], preferred_element_type=jnp.float32)
    h = h + nb_ref[...]
    h_ref[...] = h
    a_ref[...] = jnp.dot(h, wi_ref[...], preferred_element_type=jnp.float32)
    b_ref[...] = jnp.dot(h, wj_ref[...], preferred_element_type=jnp.float32)

    @pl.when(pl.program_id(0) == 0)
    def _():
        ea = ew8_ref[...] + eb_ref[...]
        t1 = jnp.dot(ea, eew1_ref[...], preferred_element_type=jnp.float32) + eeb1_ref[...]
        te1_ref[...] = jnp.dot(t1, we1_ref[...], preferred_element_type=jnp.float32) + pb1_ref[...]
        t2 = jnp.dot(ea, eew2_ref[...], preferred_element_type=jnp.float32) + eeb2_ref[...]
        te2_ref[...] = jnp.dot(t2, we2_ref[...], preferred_element_type=jnp.float32) + pb2_ref[...]


def _stats_kernel(src_ref, dst_ref, attr_ref, b_ref, te_ref,
                  s1_ref, s2_ref, mn_ref, mx_ref, deg_ref, *, with_deg):
    @pl.when(pl.program_id(0) == 0)
    def _():
        s1_ref[...] = jnp.zeros_like(s1_ref)
        s2_ref[...] = jnp.zeros_like(s2_ref)
        mn_ref[...] = jnp.full_like(mn_ref, jnp.inf)
        mx_ref[...] = jnp.full_like(mx_ref, -jnp.inf)
        if with_deg:
            deg_ref[...] = jnp.zeros_like(deg_ref)

    def body(i, carry):
        s = src_ref[0, 0, i]
        d = dst_ref[0, 0, i]
        a = attr_ref[0, 0, i]
        c = b_ref[pl.ds(s, 1), :] + te_ref[pl.ds(a, 1), :]
        s1_ref[pl.ds(d, 1), :] += c
        s2_ref[pl.ds(d, 1), :] += c * c
        mn_ref[pl.ds(d, 1), :] = jnp.minimum(mn_ref[pl.ds(d, 1), :], c)
        mx_ref[pl.ds(d, 1), :] = jnp.maximum(mx_ref[pl.ds(d, 1), :], c)
        if with_deg:
            deg_ref[pl.ds(d, 1), :] += 1.0
        return carry

    jax.lax.fori_loop(0, CE, body, 0)


def _run_stats(src3, dst3, attr3, b_tab, te, with_deg):
    full = lambda shp: pl.BlockSpec(shp, lambda i: (0,) * len(shp))
    idx_spec = pl.BlockSpec((1, 1, CE), lambda i: (i, 0, 0), memory_space=SM)
    C = T * H
    out_shapes = [jax.ShapeDtypeStruct((N, C), jnp.float32)] * 4
    out_specs = [full((N, C))] * 4
    if with_deg:
        out_shapes = out_shapes + [jax.ShapeDtypeStruct((N, 1), jnp.float32)]
        out_specs = out_specs + [full((N, 1))]
        kern = functools.partial(_stats_kernel, with_deg=True)
    else:
        def kern(src_ref, dst_ref, attr_ref, b_ref, te_ref,
                 s1_ref, s2_ref, mn_ref, mx_ref):
            _stats_kernel(src_ref, dst_ref, attr_ref, b_ref, te_ref,
                          s1_ref, s2_ref, mn_ref, mx_ref, None, with_deg=False)
    return pl.pallas_call(
        kern,
        grid=(EBLK,),
        in_specs=[idx_spec, idx_spec, idx_spec, full((N, C)), full((8, C))],
        out_specs=out_specs,
        out_shape=out_shapes,
        compiler_params=_SEQ,
    )(src3, dst3, attr3, b_tab, te)


def _node_kernel(h_ref, a_ref, s1_ref, s2_ref, mn_ref, mx_ref, deg_ref,
                 px_ref, pa_ref, pb_ref, pc_ref, postb_ref, lw_ref, lb_ref,
                 y_ref, ysum_ref, ysq_ref):
    d = deg_ref[...]
    pos = d > 0.0
    dc = jnp.maximum(d, 1.0)
    inv = 1.0 / dc
    a = a_ref[...]
    s1 = s1_ref[...] * inv
    s2 = s2_ref[...] * inv
    mean = jnp.where(pos, a + s1, 0.0)
    std = jnp.sqrt(jnp.maximum(s2 - s1 * s1, 0.0) + 1e-5)
    mn = jnp.where(pos, a + mn_ref[...], 0.0)
    mx = jnp.where(pos, a + mx_ref[...], 0.0)
    agg = jnp.concatenate([mean, mn, mx, std], axis=1)
    dl = jnp.log(dc + 1.0)
    sc1 = dl * (1.0 / AVG_LOG)
    sc2 = AVG_LOG / dl
    h = h_ref[...]
    outs = jnp.dot(h, px_ref[...], preferred_element_type=jnp.float32)
    outs += jnp.dot(agg, pa_ref[...], preferred_element_type=jnp.float32)
    outs += sc1 * jnp.dot(agg, pb_ref[...], preferred_element_type=jnp.float32)
    outs += sc2 * jnp.dot(agg, pc_ref[...], preferred_element_type=jnp.float32)
    outs += postb_ref[...]
    y = jnp.dot(outs, lw_ref[...], preferred_element_type=jnp.float32) + lb_ref[...]
    y_ref[...] = y

    @pl.when(pl.program_id(0) == 0)
    def _():
        ysum_ref[...] = jnp.zeros_like(ysum_ref)
        ysq_ref[...] = jnp.zeros_like(ysq_ref)

    ysum_ref[...] += jnp.sum(y, axis=0, keepdims=True)
    ysq_ref[...] += jnp.sum(y * y, axis=0, keepdims=True)


def _run_node(h, a_tab, s1, s2, mn, mx, deg, px, pa, pb, pc, postb, lw, lb):
    blk = lambda w: pl.BlockSpec((BN, w), lambda i: (i, 0))
    full = lambda shp: pl.BlockSpec(shp, lambda i: (0,) * len(shp))
    C = T * H
    return pl.pallas_call(
        _node_kernel,
        grid=(NBLK,),
        in_specs=[blk(H), blk(C), blk(C), blk(C), blk(C), blk(C), blk(1),
                  full((H, H)), full((4 * C, H)), full((4 * C, H)),
                  full((4 * C, H)), full((1, H)), full((H, H)), full((1, H))],
        out_specs=[blk(H), full((1, H)), full((1, H))],
        out_shape=[jax.ShapeDtypeStruct((N, H), jnp.float32),
                   jax.ShapeDtypeStruct((1, H), jnp.float32),
                   jax.ShapeDtypeStruct((1, H), jnp.float32)],
        compiler_params=_SEQ,
    )(h, a_tab, s1, s2, mn, mx, deg, px, pa, pb, pc, postb, lw, lb)


def _mid_kernel(y_ref, ysum_ref, ysq_ref, g_ref, bb_ref, h0_ref,
                wi_ref, wj_ref, h1_ref, a_ref, b_ref):
    m = ysum_ref[...] * (1.0 / N)
    v = ysq_ref[...] * (1.0 / N) - m * m
    inv = jax.lax.rsqrt(v + 1e-5)
    h1 = jnp.maximum((y_ref[...] - m) * inv * g_ref[...] + bb_ref[...], 0.0)
    h1 = h1 + h0_ref[...]
    h1_ref[...] = h1
    a_ref[...] = jnp.dot(h1, wi_ref[...], preferred_element_type=jnp.float32)
    b_ref[...] = jnp.dot(h1, wj_ref[...], preferred_element_type=jnp.float32)


def _head_kernel(y_ref, ysum_ref, ysq_ref, g_ref, bb_ref, h1_ref, batch_ref,
                 yt_ref, l1w_ref, l1b_ref, l3w_ref, l3b_ref,
                 xcat_ref, loss_ref):
    m = ysum_ref[...] * (1.0 / N)
    v = ysq_ref[...] * (1.0 / N) - m * m
    inv = jax.lax.rsqrt(v + 1e-5)
    h2 = jnp.maximum((y_ref[...] - m) * inv * g_ref[...] + bb_ref[...], 0.0)
    h2 = h2 + h1_ref[...]
    oh = (batch_ref[...] ==
          jax.lax.broadcasted_iota(jnp.int32, (BN, G), 1)).astype(jnp.float32)

    @pl.when(pl.program_id(0) == 0)
    def _():
        xcat_ref[...] = jnp.zeros_like(xcat_ref)

    xcat_ref[...] += jax.lax.dot_general(
        oh, h2, (((0,), (0,)), ((), ())), preferred_element_type=jnp.float32)

    @pl.when(pl.program_id(0) == NBLK - 1)
    def _():
        xc = jnp.maximum(
            jnp.dot(xcat_ref[...], l1w_ref[...], preferred_element_type=jnp.float32)
            + l1b_ref[...], 0.0)
        p = jnp.dot(xc, l3w_ref[...], preferred_element_type=jnp.float32) + l3b_ref[...]
        dd = jnp.abs(p - yt_ref[...])
        lv = jnp.where(dd < 0.5, dd * dd, dd - 0.25)
        loss_ref[...] = jnp.sum(lv, keepdims=True).reshape(1, 1) * (1.0 / G)


def kernel(x, edge_index, edge_attr, batch, y, node_W, node_b, eenc_W, eenc_b,
           pna_ee_W, pna_ee_b, pre_W, pre_b, post_W, post_b, lin_W, lin_b,
           bn_g, bn_b, lin1_W, lin1_b, lin3_W, lin3_b):
    f32 = jnp.float32
    C = T * H
    src3 = edge_index[0].reshape(EBLK, 1, CE)
    dst3 = edge_index[1].reshape(EBLK, 1, CE)
    attr3 = edge_attr.reshape(EBLK, 1, CE)
    batch2 = batch.reshape(N, 1)
    yt = y.reshape(G, 1)

    wi, wj, we, pbv, px, pabc, postb = [], [], [], [], [], [], []
    eye4 = jnp.eye(T, dtype=f32)
    for l in range(L):
        pw = pre_W[l]
        wi.append(pw[:, :H, :].transpose(1, 0, 2).reshape(H, C))
        wj.append(pw[:, H:2 * H, :].transpose(1, 0, 2).reshape(H, C))
        we.append(pw[:, 2 * H:, :].transpose(1, 0, 2).reshape(H, C))
        pbv.append(pre_b[l].reshape(1, C))
        po = post_W[l]
        px.append(po[:, :H, :].transpose(1, 0, 2).reshape(H, H))
        pw2 = po[:, H:, :].reshape(T, 12, H, H // T)
        groups = []
        for g in range(3):
            bg = pw2[:, g * 4:(g + 1) * 4].transpose(1, 0, 2, 3)
            z = jnp.einsum('stoj,tu->stouj', bg, eye4)
            groups.append(z.reshape(4 * C, H))
        pabc.append(groups)
        postb.append(post_b[l].reshape(1, H))
    eenc_W8 = jnp.pad(eenc_W, ((0, 4), (0, 0)))
    eenc_b2 = eenc_b.reshape(1, H)
    te_pb = []
    for l in range(L):
        te_pb.append((pna_ee_W[l], pna_ee_b[l].reshape(1, H), we[l], pbv[l]))

    blk = lambda w: pl.BlockSpec((BN, w), lambda i: (i, 0))
    full = lambda shp: pl.BlockSpec(shp, lambda i: (0,) * len(shp))
    h0, a1, b1, te1, te2 = pl.pallas_call(
        _enc_kernel,
        grid=(NBLK,),
        in_specs=[blk(9), full((9, H)), full((1, H)), full((H, C)), full((H, C)),
                  full((8, H)), full((1, H)),
                  full((H, H)), full((1, H)), full((H, C)), full((1, C)),
                  full((H, H)), full((1, H)), full((H, C)), full((1, C))],
        out_specs=[blk(H), blk(C), blk(C), full((8, C)), full((8, C))],
        out_shape=[jax.ShapeDtypeStruct((N, H), f32),
                   jax.ShapeDtypeStruct((N, C), f32),
                   jax.ShapeDtypeStruct((N, C), f32),
                   jax.ShapeDtypeStruct((8, C), f32),
                   jax.ShapeDtypeStruct((8, C), f32)],
        compiler_params=_SEQ,
    )(x, node_W, node_b.reshape(1, H), wi[0], wj[0],
      eenc_W8, eenc_b2, te_pb[0][0], te_pb[0][1], te_pb[0][2], te_pb[0][3],
      te_pb[1][0], te_pb[1][1], te_pb[1][2], te_pb[1][3])

    s1, s2, mn, mx, deg = _run_stats(src3, dst3, attr3, b1, te1, with_deg=True)
    y1, ysum1, ysq1 = _run_node(h0, a1, s1, s2, mn, mx, deg,
                                px[0], pabc[0][0], pabc[0][1], pabc[0][2],
                                postb[0], lin_W[0], lin_b[0].reshape(1, H))

    h1, a2, b2 = pl.pallas_call(
        _mid_kernel,
        grid=(NBLK,),
        in_specs=[blk(H), full((1, H)), full((1, H)), full((1, H)), full((1, H)),
                  blk(H), full((H, C)), full((H, C))],
        out_specs=[blk(H), blk(C), blk(C)],
        out_shape=[jax.ShapeDtypeStruct((N, H), f32),
                   jax.ShapeDtypeStruct((N, C), f32),
                   jax.ShapeDtypeStruct((N, C), f32)],
        compiler_params=_SEQ,
    )(y1, ysum1, ysq1, bn_g[0].reshape(1, H), bn_b[0].reshape(1, H),
      h0, wi[1], wj[1])

    s1b, s2b, mnb, mxb = _run_stats(src3, dst3, attr3, b2, te2, with_deg=False)
    y2, ysum2, ysq2 = _run_node(h1, a2, s1b, s2b, mnb, mxb, deg,
                                px[1], pabc[1][0], pabc[1][1], pabc[1][2],
                                postb[1], lin_W[1], lin_b[1].reshape(1, H))

    _, loss2d = pl.pallas_call(
        _head_kernel,
        grid=(NBLK,),
        in_specs=[blk(H), full((1, H)), full((1, H)), full((1, H)), full((1, H)),
                  blk(H), pl.BlockSpec((BN, 1), lambda i: (i, 0)),
                  full((G, 1)), full((H, H)), full((1, H)), full((H, 1)),
                  full((1, 1))],
        out_specs=[full((G, H)), full((1, 1))],
        out_shape=[jax.ShapeDtypeStruct((G, H), f32),
                   jax.ShapeDtypeStruct((1, 1), f32)],
        compiler_params=_SEQ,
    )(y2, ysum2, ysq2, bn_g[1].reshape(1, H), bn_b[1].reshape(1, H),
      h1, batch2, yt, lin1_W, lin1_b.reshape(1, H), lin3_W,
      lin3_b.reshape(1, 1))

    loss = loss2d.reshape(())
    return (loss, loss)


# speedup vs baseline: 2.0767x; 1.5780x over previous
import functools

import jax
import jax.numpy as jnp
import numpy as np
from jax.experimental import pallas as pl
from jax.experimental.pallas import tpu as pltpu

N = 10000
E = 160000
H = 64
L = 2
T = 4
G = 128
AVG_LOG = float(np.mean(np.log(np.arange(10, dtype=np.float64) + 1.0)))

BN = 1000
CE = 2000
NBLK = N // BN
EBLK = E // CE
SM = pltpu.MemorySpace.SMEM
_SEQ = pltpu.CompilerParams(dimension_semantics=("arbitrary",))


def _enc_kernel(x_ref, nw_ref, nb_ref, wi_ref, wj_ref,
                ew8_ref, eb_ref, eew1_ref, eeb1_ref, we1_ref, pb1_ref,
                eew2_ref, eeb2_ref, we2_ref, pb2_ref,
                h_ref, a_ref, b_ref, te1_ref, te2_ref):
    h = jnp.dot(x_ref[...], nw_ref[...], preferred_element_type=jnp.float32)
    h = h + nb_ref[...]
    h_ref[...] = h
    a_ref[...] = jnp.dot(h, wi_ref[...], preferred_element_type=jnp.float32)
    b_ref[...] = jnp.dot(h, wj_ref[...], preferred_element_type=jnp.float32)

    @pl.when(pl.program_id(0) == 0)
    def _():
        ea = ew8_ref[...] + eb_ref[...]
        t1 = jnp.dot(ea, eew1_ref[...], preferred_element_type=jnp.float32) + eeb1_ref[...]
        te1_ref[...] = jnp.dot(t1, we1_ref[...], preferred_element_type=jnp.float32) + pb1_ref[...]
        t2 = jnp.dot(ea, eew2_ref[...], preferred_element_type=jnp.float32) + eeb2_ref[...]
        te2_ref[...] = jnp.dot(t2, we2_ref[...], preferred_element_type=jnp.float32) + pb2_ref[...]


def _stats_kernel(src_ref, dst_ref, attr_ref, b_ref, te_ref,
                  s1_ref, s2_ref, mn_ref, mx_ref, deg_ref, *, with_deg):
    @pl.when(pl.program_id(0) == 0)
    def _():
        s1_ref[...] = jnp.zeros_like(s1_ref)
        s2_ref[...] = jnp.zeros_like(s2_ref)
        mn_ref[...] = jnp.full_like(mn_ref, jnp.inf)
        mx_ref[...] = jnp.full_like(mx_ref, -jnp.inf)
        if with_deg:
            deg_ref[...] = jnp.zeros_like(deg_ref)

    U = 4

    def body(i, carry):
        base = i * U
        ds, cs = [], []
        for u in range(U):
            s = src_ref[0, 0, base + u]
            a = attr_ref[0, 0, base + u]
            ds.append(dst_ref[0, 0, base + u])
            cs.append(b_ref[pl.ds(s, 1), :] + te_ref[pl.ds(a, 1), :])
        for u in range(U):
            s1_ref[pl.ds(ds[u], 1), :] += cs[u]
        for u in range(U):
            s2_ref[pl.ds(ds[u], 1), :] += cs[u] * cs[u]
        for u in range(U):
            mn_ref[pl.ds(ds[u], 1), :] = jnp.minimum(
                mn_ref[pl.ds(ds[u], 1), :], cs[u])
        for u in range(U):
            mx_ref[pl.ds(ds[u], 1), :] = jnp.maximum(
                mx_ref[pl.ds(ds[u], 1), :], cs[u])
        if with_deg:
            for u in range(U):
                deg_ref[pl.ds(ds[u], 1), :] += 1.0
        return carry

    jax.lax.fori_loop(0, CE // U, body, 0)


def _run_stats(src3, dst3, attr3, b_tab, te, with_deg):
    full = lambda shp: pl.BlockSpec(shp, lambda i: (0,) * len(shp))
    idx_spec = pl.BlockSpec((1, 1, CE), lambda i: (i, 0, 0), memory_space=SM)
    C = T * H
    out_shapes = [jax.ShapeDtypeStruct((N, C), jnp.float32)] * 4
    out_specs = [full((N, C))] * 4
    if with_deg:
        out_shapes = out_shapes + [jax.ShapeDtypeStruct((N, 1), jnp.float32)]
        out_specs = out_specs + [full((N, 1))]
        kern = functools.partial(_stats_kernel, with_deg=True)
    else:
        def kern(src_ref, dst_ref, attr_ref, b_ref, te_ref,
                 s1_ref, s2_ref, mn_ref, mx_ref):
            _stats_kernel(src_ref, dst_ref, attr_ref, b_ref, te_ref,
                          s1_ref, s2_ref, mn_ref, mx_ref, None, with_deg=False)
    return pl.pallas_call(
        kern,
        grid=(EBLK,),
        in_specs=[idx_spec, idx_spec, idx_spec, full((N, C)), full((8, C))],
        out_specs=out_specs,
        out_shape=out_shapes,
        compiler_params=_SEQ,
    )(src3, dst3, attr3, b_tab, te)


def _node_kernel(h_ref, a_ref, s1_ref, s2_ref, mn_ref, mx_ref, deg_ref,
                 px_ref, pa_ref, pb_ref, pc_ref, postb_ref, lw_ref, lb_ref,
                 y_ref, ysum_ref, ysq_ref):
    d = deg_ref[...]
    pos = d > 0.0
    dc = jnp.maximum(d, 1.0)
    inv = 1.0 / dc
    a = a_ref[...]
    s1 = s1_ref[...] * inv
    s2 = s2_ref[...] * inv
    mean = jnp.where(pos, a + s1, 0.0)
    std = jnp.sqrt(jnp.maximum(s2 - s1 * s1, 0.0) + 1e-5)
    mn = jnp.where(pos, a + mn_ref[...], 0.0)
    mx = jnp.where(pos, a + mx_ref[...], 0.0)
    agg = jnp.concatenate([mean, mn, mx, std], axis=1)
    dl = jnp.log(dc + 1.0)
    sc1 = dl * (1.0 / AVG_LOG)
    sc2 = AVG_LOG / dl
    h = h_ref[...]
    outs = jnp.dot(h, px_ref[...], preferred_element_type=jnp.float32)
    outs += jnp.dot(agg, pa_ref[...], preferred_element_type=jnp.float32)
    outs += sc1 * jnp.dot(agg, pb_ref[...], preferred_element_type=jnp.float32)
    outs += sc2 * jnp.dot(agg, pc_ref[...], preferred_element_type=jnp.float32)
    outs += postb_ref[...]
    y = jnp.dot(outs, lw_ref[...], preferred_element_type=jnp.float32) + lb_ref[...]
    y_ref[...] = y

    @pl.when(pl.program_id(0) == 0)
    def _():
        ysum_ref[...] = jnp.zeros_like(ysum_ref)
        ysq_ref[...] = jnp.zeros_like(ysq_ref)

    ysum_ref[...] += jnp.sum(y, axis=0, keepdims=True)
    ysq_ref[...] += jnp.sum(y * y, axis=0, keepdims=True)


def _run_node(h, a_tab, s1, s2, mn, mx, deg, px, pa, pb, pc, postb, lw, lb):
    blk = lambda w: pl.BlockSpec((BN, w), lambda i: (i, 0))
    full = lambda shp: pl.BlockSpec(shp, lambda i: (0,) * len(shp))
    C = T * H
    return pl.pallas_call(
        _node_kernel,
        grid=(NBLK,),
        in_specs=[blk(H), blk(C), blk(C), blk(C), blk(C), blk(C), blk(1),
                  full((H, H)), full((4 * C, H)), full((4 * C, H)),
                  full((4 * C, H)), full((1, H)), full((H, H)), full((1, H))],
        out_specs=[blk(H), full((1, H)), full((1, H))],
        out_shape=[jax.ShapeDtypeStruct((N, H), jnp.float32),
                   jax.ShapeDtypeStruct((1, H), jnp.float32),
                   jax.ShapeDtypeStruct((1, H), jnp.float32)],
        compiler_params=_SEQ,
    )(h, a_tab, s1, s2, mn, mx, deg, px, pa, pb, pc, postb, lw, lb)


def _mid_kernel(y_ref, ysum_ref, ysq_ref, g_ref, bb_ref, h0_ref,
                wi_ref, wj_ref, h1_ref, a_ref, b_ref):
    m = ysum_ref[...] * (1.0 / N)
    v = ysq_ref[...] * (1.0 / N) - m * m
    inv = jax.lax.rsqrt(v + 1e-5)
    h1 = jnp.maximum((y_ref[...] - m) * inv * g_ref[...] + bb_ref[...], 0.0)
    h1 = h1 + h0_ref[...]
    h1_ref[...] = h1
    a_ref[...] = jnp.dot(h1, wi_ref[...], preferred_element_type=jnp.float32)
    b_ref[...] = jnp.dot(h1, wj_ref[...], preferred_element_type=jnp.float32)


def _head_kernel(y_ref, ysum_ref, ysq_ref, g_ref, bb_ref, h1_ref, batch_ref,
                 yt_ref, l1w_ref, l1b_ref, l3w_ref, l3b_ref,
                 xcat_ref, loss_ref):
    m = ysum_ref[...] * (1.0 / N)
    v = ysq_ref[...] * (1.0 / N) - m * m
    inv = jax.lax.rsqrt(v + 1e-5)
    h2 = jnp.maximum((y_ref[...] - m) * inv * g_ref[...] + bb_ref[...], 0.0)
    h2 = h2 + h1_ref[...]
    oh = (batch_ref[...] ==
          jax.lax.broadcasted_iota(jnp.int32, (BN, G), 1)).astype(jnp.float32)

    @pl.when(pl.program_id(0) == 0)
    def _():
        xcat_ref[...] = jnp.zeros_like(xcat_ref)

    xcat_ref[...] += jax.lax.dot_general(
        oh, h2, (((0,), (0,)), ((), ())), preferred_element_type=jnp.float32)

    @pl.when(pl.program_id(0) == NBLK - 1)
    def _():
        xc = jnp.maximum(
            jnp.dot(xcat_ref[...], l1w_ref[...], preferred_element_type=jnp.float32)
            + l1b_ref[...], 0.0)
        p = jnp.dot(xc, l3w_ref[...], preferred_element_type=jnp.float32) + l3b_ref[...]
        dd = jnp.abs(p - yt_ref[...])
        lv = jnp.where(dd < 0.5, dd * dd, dd - 0.25)
        loss_ref[...] = jnp.sum(lv, keepdims=True).reshape(1, 1) * (1.0 / G)


def kernel(x, edge_index, edge_attr, batch, y, node_W, node_b, eenc_W, eenc_b,
           pna_ee_W, pna_ee_b, pre_W, pre_b, post_W, post_b, lin_W, lin_b,
           bn_g, bn_b, lin1_W, lin1_b, lin3_W, lin3_b):
    f32 = jnp.float32
    C = T * H
    src3 = edge_index[0].reshape(EBLK, 1, CE)
    dst3 = edge_index[1].reshape(EBLK, 1, CE)
    attr3 = edge_attr.reshape(EBLK, 1, CE)
    batch2 = batch.reshape(N, 1)
    yt = y.reshape(G, 1)

    wi, wj, we, pbv, px, pabc, postb = [], [], [], [], [], [], []
    eye4 = jnp.eye(T, dtype=f32)
    for l in range(L):
        pw = pre_W[l]
        wi.append(pw[:, :H, :].transpose(1, 0, 2).reshape(H, C))
        wj.append(pw[:, H:2 * H, :].transpose(1, 0, 2).reshape(H, C))
        we.append(pw[:, 2 * H:, :].transpose(1, 0, 2).reshape(H, C))
        pbv.append(pre_b[l].reshape(1, C))
        po = post_W[l]
        px.append(po[:, :H, :].transpose(1, 0, 2).reshape(H, H))
        pw2 = po[:, H:, :].reshape(T, 12, H, H // T)
        groups = []
        for g in range(3):
            bg = pw2[:, g * 4:(g + 1) * 4].transpose(1, 0, 2, 3)
            z = jnp.einsum('stoj,tu->stouj', bg, eye4)
            groups.append(z.reshape(4 * C, H))
        pabc.append(groups)
        postb.append(post_b[l].reshape(1, H))
    eenc_W8 = jnp.pad(eenc_W, ((0, 4), (0, 0)))
    eenc_b2 = eenc_b.reshape(1, H)
    te_pb = []
    for l in range(L):
        te_pb.append((pna_ee_W[l], pna_ee_b[l].reshape(1, H), we[l], pbv[l]))

    blk = lambda w: pl.BlockSpec((BN, w), lambda i: (i, 0))
    full = lambda shp: pl.BlockSpec(shp, lambda i: (0,) * len(shp))
    h0, a1, b1, te1, te2 = pl.pallas_call(
        _enc_kernel,
        grid=(NBLK,),
        in_specs=[blk(9), full((9, H)), full((1, H)), full((H, C)), full((H, C)),
                  full((8, H)), full((1, H)),
                  full((H, H)), full((1, H)), full((H, C)), full((1, C)),
                  full((H, H)), full((1, H)), full((H, C)), full((1, C))],
        out_specs=[blk(H), blk(C), blk(C), full((8, C)), full((8, C))],
        out_shape=[jax.ShapeDtypeStruct((N, H), f32),
                   jax.ShapeDtypeStruct((N, C), f32),
                   jax.ShapeDtypeStruct((N, C), f32),
                   jax.ShapeDtypeStruct((8, C), f32),
                   jax.ShapeDtypeStruct((8, C), f32)],
        compiler_params=_SEQ,
    )(x, node_W, node_b.reshape(1, H), wi[0], wj[0],
      eenc_W8, eenc_b2, te_pb[0][0], te_pb[0][1], te_pb[0][2], te_pb[0][3],
      te_pb[1][0], te_pb[1][1], te_pb[1][2], te_pb[1][3])

    s1, s2, mn, mx, deg = _run_stats(src3, dst3, attr3, b1, te1, with_deg=True)
    y1, ysum1, ysq1 = _run_node(h0, a1, s1, s2, mn, mx, deg,
                                px[0], pabc[0][0], pabc[0][1], pabc[0][2],
                                postb[0], lin_W[0], lin_b[0].reshape(1, H))

    h1, a2, b2 = pl.pallas_call(
        _mid_kernel,
        grid=(NBLK,),
        in_specs=[blk(H), full((1, H)), full((1, H)), full((1, H)), full((1, H)),
                  blk(H), full((H, C)), full((H, C))],
        out_specs=[blk(H), blk(C), blk(C)],
        out_shape=[jax.ShapeDtypeStruct((N, H), f32),
                   jax.ShapeDtypeStruct((N, C), f32),
                   jax.ShapeDtypeStruct((N, C), f32)],
        compiler_params=_SEQ,
    )(y1, ysum1, ysq1, bn_g[0].reshape(1, H), bn_b[0].reshape(1, H),
      h0, wi[1], wj[1])

    s1b, s2b, mnb, mxb = _run_stats(src3, dst3, attr3, b2, te2, with_deg=False)
    y2, ysum2, ysq2 = _run_node(h1, a2, s1b, s2b, mnb, mxb, deg,
                                px[1], pabc[1][0], pabc[1][1], pabc[1][2],
                                postb[1], lin_W[1], lin_b[1].reshape(1, H))

    _, loss2d = pl.pallas_call(
        _head_kernel,
        grid=(NBLK,),
        in_specs=[blk(H), full((1, H)), full((1, H)), full((1, H)), full((1, H)),
                  blk(H), pl.BlockSpec((BN, 1), lambda i: (i, 0)),
                  full((G, 1)), full((H, H)), full((1, H)), full((H, 1)),
                  full((1, 1))],
        out_specs=[full((G, H)), full((1, 1))],
        out_shape=[jax.ShapeDtypeStruct((G, H), f32),
                   jax.ShapeDtypeStruct((1, 1), f32)],
        compiler_params=_SEQ,
    )(y2, ysum2, ysq2, bn_g[1].reshape(1, H), bn_b[1].reshape(1, H),
      h1, batch2, yt, lin1_W, lin1_b.reshape(1, H), lin3_W,
      lin3_b.reshape(1, 1))

    loss = loss2d.reshape(())
    return (loss, loss)


# speedup vs baseline: 2.2061x; 1.0623x over previous
import functools

import jax
import jax.numpy as jnp
import numpy as np
from jax.experimental import pallas as pl
from jax.experimental.pallas import tpu as pltpu

N = 10000
E = 160000
H = 64
L = 2
T = 4
G = 128
AVG_LOG = float(np.mean(np.log(np.arange(10, dtype=np.float64) + 1.0)))

BN = 1000
CE = 2000
NBLK = N // BN
EBLK = E // CE
SM = pltpu.MemorySpace.SMEM
_SEQ = pltpu.CompilerParams(dimension_semantics=("arbitrary",))


def _enc_kernel(x_ref, nw_ref, nb_ref, wi_ref, wj_ref,
                ew8_ref, eb_ref, eew1_ref, eeb1_ref, we1_ref, pb1_ref,
                eew2_ref, eeb2_ref, we2_ref, pb2_ref,
                h_ref, a_ref, b_ref, te1_ref, te2_ref):
    h = jnp.dot(x_ref[...], nw_ref[...], preferred_element_type=jnp.float32)
    h = h + nb_ref[...]
    h_ref[...] = h
    a_ref[...] = jnp.dot(h, wi_ref[...], preferred_element_type=jnp.float32)
    b_ref[...] = jnp.dot(h, wj_ref[...], preferred_element_type=jnp.float32)

    @pl.when(pl.program_id(0) == 0)
    def _():
        ea = ew8_ref[...] + eb_ref[...]
        t1 = jnp.dot(ea, eew1_ref[...], preferred_element_type=jnp.float32) + eeb1_ref[...]
        te1_ref[...] = jnp.dot(t1, we1_ref[...], preferred_element_type=jnp.float32) + pb1_ref[...]
        t2 = jnp.dot(ea, eew2_ref[...], preferred_element_type=jnp.float32) + eeb2_ref[...]
        te2_ref[...] = jnp.dot(t2, we2_ref[...], preferred_element_type=jnp.float32) + pb2_ref[...]


def _stats_kernel(src_ref, dst_ref, attr_ref, b_ref, te_ref,
                  s1_ref, s2_ref, mn_ref, mx_ref, deg_ref, *, with_deg):
    @pl.when(pl.program_id(0) == 0)
    def _():
        s1_ref[...] = jnp.zeros_like(s1_ref)
        s2_ref[...] = jnp.zeros_like(s2_ref)
        mn_ref[...] = jnp.full_like(mn_ref, jnp.inf)
        mx_ref[...] = jnp.full_like(mx_ref, -jnp.inf)
        if with_deg:
            deg_ref[...] = jnp.zeros_like(deg_ref)

    U = 8

    def body(i, carry):
        base = i * U
        ds, cs = [], []
        for u in range(U):
            s = src_ref[0, 0, base + u]
            a = attr_ref[0, 0, base + u]
            ds.append(dst_ref[0, 0, base + u])
            cs.append(b_ref[pl.ds(s, 1), :] + te_ref[pl.ds(a, 1), :])
        for u in range(U):
            s1_ref[pl.ds(ds[u], 1), :] += cs[u]
        for u in range(U):
            s2_ref[pl.ds(ds[u], 1), :] += cs[u] * cs[u]
        for u in range(U):
            mn_ref[pl.ds(ds[u], 1), :] = jnp.minimum(
                mn_ref[pl.ds(ds[u], 1), :], cs[u])
        for u in range(U):
            mx_ref[pl.ds(ds[u], 1), :] = jnp.maximum(
                mx_ref[pl.ds(ds[u], 1), :], cs[u])
        if with_deg:
            for u in range(U):
                deg_ref[pl.ds(ds[u], 1), :] += 1.0
        return carry

    jax.lax.fori_loop(0, CE // U, body, 0)


def _run_stats(src3, dst3, attr3, b_tab, te, with_deg):
    full = lambda shp: pl.BlockSpec(shp, lambda i: (0,) * len(shp))
    idx_spec = pl.BlockSpec((1, 1, CE), lambda i: (i, 0, 0), memory_space=SM)
    C = T * H
    out_shapes = [jax.ShapeDtypeStruct((N, C), jnp.float32)] * 4
    out_specs = [full((N, C))] * 4
    if with_deg:
        out_shapes = out_shapes + [jax.ShapeDtypeStruct((N, 1), jnp.float32)]
        out_specs = out_specs + [full((N, 1))]
        kern = functools.partial(_stats_kernel, with_deg=True)
    else:
        def kern(src_ref, dst_ref, attr_ref, b_ref, te_ref,
                 s1_ref, s2_ref, mn_ref, mx_ref):
            _stats_kernel(src_ref, dst_ref, attr_ref, b_ref, te_ref,
                          s1_ref, s2_ref, mn_ref, mx_ref, None, with_deg=False)
    return pl.pallas_call(
        kern,
        grid=(EBLK,),
        in_specs=[idx_spec, idx_spec, idx_spec, full((N, C)), full((8, C))],
        out_specs=out_specs,
        out_shape=out_shapes,
        compiler_params=_SEQ,
    )(src3, dst3, attr3, b_tab, te)


def _node_kernel(h_ref, a_ref, s1_ref, s2_ref, mn_ref, mx_ref, deg_ref,
                 px_ref, pa_ref, pb_ref, pc_ref, postb_ref, lw_ref, lb_ref,
                 y_ref, ysum_ref, ysq_ref):
    d = deg_ref[...]
    pos = d > 0.0
    dc = jnp.maximum(d, 1.0)
    inv = 1.0 / dc
    a = a_ref[...]
    s1 = s1_ref[...] * inv
    s2 = s2_ref[...] * inv
    mean = jnp.where(pos, a + s1, 0.0)
    std = jnp.sqrt(jnp.maximum(s2 - s1 * s1, 0.0) + 1e-5)
    mn = jnp.where(pos, a + mn_ref[...], 0.0)
    mx = jnp.where(pos, a + mx_ref[...], 0.0)
    agg = jnp.concatenate([mean, mn, mx, std], axis=1)
    dl = jnp.log(dc + 1.0)
    sc1 = dl * (1.0 / AVG_LOG)
    sc2 = AVG_LOG / dl
    h = h_ref[...]
    outs = jnp.dot(h, px_ref[...], preferred_element_type=jnp.float32)
    outs += jnp.dot(agg, pa_ref[...], preferred_element_type=jnp.float32)
    outs += sc1 * jnp.dot(agg, pb_ref[...], preferred_element_type=jnp.float32)
    outs += sc2 * jnp.dot(agg, pc_ref[...], preferred_element_type=jnp.float32)
    outs += postb_ref[...]
    y = jnp.dot(outs, lw_ref[...], preferred_element_type=jnp.float32) + lb_ref[...]
    y_ref[...] = y

    @pl.when(pl.program_id(0) == 0)
    def _():
        ysum_ref[...] = jnp.zeros_like(ysum_ref)
        ysq_ref[...] = jnp.zeros_like(ysq_ref)

    ysum_ref[...] += jnp.sum(y, axis=0, keepdims=True)
    ysq_ref[...] += jnp.sum(y * y, axis=0, keepdims=True)


def _run_node(h, a_tab, s1, s2, mn, mx, deg, px, pa, pb, pc, postb, lw, lb):
    blk = lambda w: pl.BlockSpec((BN, w), lambda i: (i, 0))
    full = lambda shp: pl.BlockSpec(shp, lambda i: (0,) * len(shp))
    C = T * H
    return pl.pallas_call(
        _node_kernel,
        grid=(NBLK,),
        in_specs=[blk(H), blk(C), blk(C), blk(C), blk(C), blk(C), blk(1),
                  full((H, H)), full((4 * C, H)), full((4 * C, H)),
                  full((4 * C, H)), full((1, H)), full((H, H)), full((1, H))],
        out_specs=[blk(H), full((1, H)), full((1, H))],
        out_shape=[jax.ShapeDtypeStruct((N, H), jnp.float32),
                   jax.ShapeDtypeStruct((1, H), jnp.float32),
                   jax.ShapeDtypeStruct((1, H), jnp.float32)],
        compiler_params=_SEQ,
    )(h, a_tab, s1, s2, mn, mx, deg, px, pa, pb, pc, postb, lw, lb)


def _mid_kernel(y_ref, ysum_ref, ysq_ref, g_ref, bb_ref, h0_ref,
                wi_ref, wj_ref, h1_ref, a_ref, b_ref):
    m = ysum_ref[...] * (1.0 / N)
    v = ysq_ref[...] * (1.0 / N) - m * m
    inv = jax.lax.rsqrt(v + 1e-5)
    h1 = jnp.maximum((y_ref[...] - m) * inv * g_ref[...] + bb_ref[...], 0.0)
    h1 = h1 + h0_ref[...]
    h1_ref[...] = h1
    a_ref[...] = jnp.dot(h1, wi_ref[...], preferred_element_type=jnp.float32)
    b_ref[...] = jnp.dot(h1, wj_ref[...], preferred_element_type=jnp.float32)


def _head_kernel(y_ref, ysum_ref, ysq_ref, g_ref, bb_ref, h1_ref, batch_ref,
                 yt_ref, l1w_ref, l1b_ref, l3w_ref, l3b_ref,
                 xcat_ref, loss_ref):
    m = ysum_ref[...] * (1.0 / N)
    v = ysq_ref[...] * (1.0 / N) - m * m
    inv = jax.lax.rsqrt(v + 1e-5)
    h2 = jnp.maximum((y_ref[...] - m) * inv * g_ref[...] + bb_ref[...], 0.0)
    h2 = h2 + h1_ref[...]
    oh = (batch_ref[...] ==
          jax.lax.broadcasted_iota(jnp.int32, (BN, G), 1)).astype(jnp.float32)

    @pl.when(pl.program_id(0) == 0)
    def _():
        xcat_ref[...] = jnp.zeros_like(xcat_ref)

    xcat_ref[...] += jax.lax.dot_general(
        oh, h2, (((0,), (0,)), ((), ())), preferred_element_type=jnp.float32)

    @pl.when(pl.program_id(0) == NBLK - 1)
    def _():
        xc = jnp.maximum(
            jnp.dot(xcat_ref[...], l1w_ref[...], preferred_element_type=jnp.float32)
            + l1b_ref[...], 0.0)
        p = jnp.dot(xc, l3w_ref[...], preferred_element_type=jnp.float32) + l3b_ref[...]
        dd = jnp.abs(p - yt_ref[...])
        lv = jnp.where(dd < 0.5, dd * dd, dd - 0.25)
        loss_ref[...] = jnp.sum(lv, keepdims=True).reshape(1, 1) * (1.0 / G)


def kernel(x, edge_index, edge_attr, batch, y, node_W, node_b, eenc_W, eenc_b,
           pna_ee_W, pna_ee_b, pre_W, pre_b, post_W, post_b, lin_W, lin_b,
           bn_g, bn_b, lin1_W, lin1_b, lin3_W, lin3_b):
    f32 = jnp.float32
    C = T * H
    src3 = edge_index[0].reshape(EBLK, 1, CE)
    dst3 = edge_index[1].reshape(EBLK, 1, CE)
    attr3 = edge_attr.reshape(EBLK, 1, CE)
    batch2 = batch.reshape(N, 1)
    yt = y.reshape(G, 1)

    wi, wj, we, pbv, px, pabc, postb = [], [], [], [], [], [], []
    eye4 = jnp.eye(T, dtype=f32)
    for l in range(L):
        pw = pre_W[l]
        wi.append(pw[:, :H, :].transpose(1, 0, 2).reshape(H, C))
        wj.append(pw[:, H:2 * H, :].transpose(1, 0, 2).reshape(H, C))
        we.append(pw[:, 2 * H:, :].transpose(1, 0, 2).reshape(H, C))
        pbv.append(pre_b[l].reshape(1, C))
        po = post_W[l]
        px.append(po[:, :H, :].transpose(1, 0, 2).reshape(H, H))
        pw2 = po[:, H:, :].reshape(T, 12, H, H // T)
        groups = []
        for g in range(3):
            bg = pw2[:, g * 4:(g + 1) * 4].transpose(1, 0, 2, 3)
            z = jnp.einsum('stoj,tu->stouj', bg, eye4)
            groups.append(z.reshape(4 * C, H))
        pabc.append(groups)
        postb.append(post_b[l].reshape(1, H))
    eenc_W8 = jnp.pad(eenc_W, ((0, 4), (0, 0)))
    eenc_b2 = eenc_b.reshape(1, H)
    te_pb = []
    for l in range(L):
        te_pb.append((pna_ee_W[l], pna_ee_b[l].reshape(1, H), we[l], pbv[l]))

    blk = lambda w: pl.BlockSpec((BN, w), lambda i: (i, 0))
    full = lambda shp: pl.BlockSpec(shp, lambda i: (0,) * len(shp))
    h0, a1, b1, te1, te2 = pl.pallas_call(
        _enc_kernel,
        grid=(NBLK,),
        in_specs=[blk(9), full((9, H)), full((1, H)), full((H, C)), full((H, C)),
                  full((8, H)), full((1, H)),
                  full((H, H)), full((1, H)), full((H, C)), full((1, C)),
                  full((H, H)), full((1, H)), full((H, C)), full((1, C))],
        out_specs=[blk(H), blk(C), blk(C), full((8, C)), full((8, C))],
        out_shape=[jax.ShapeDtypeStruct((N, H), f32),
                   jax.ShapeDtypeStruct((N, C), f32),
                   jax.ShapeDtypeStruct((N, C), f32),
                   jax.ShapeDtypeStruct((8, C), f32),
                   jax.ShapeDtypeStruct((8, C), f32)],
        compiler_params=_SEQ,
    )(x, node_W, node_b.reshape(1, H), wi[0], wj[0],
      eenc_W8, eenc_b2, te_pb[0][0], te_pb[0][1], te_pb[0][2], te_pb[0][3],
      te_pb[1][0], te_pb[1][1], te_pb[1][2], te_pb[1][3])

    s1, s2, mn, mx, deg = _run_stats(src3, dst3, attr3, b1, te1, with_deg=True)
    y1, ysum1, ysq1 = _run_node(h0, a1, s1, s2, mn, mx, deg,
                                px[0], pabc[0][0], pabc[0][1], pabc[0][2],
                                postb[0], lin_W[0], lin_b[0].reshape(1, H))

    h1, a2, b2 = pl.pallas_call(
        _mid_kernel,
        grid=(NBLK,),
        in_specs=[blk(H), full((1, H)), full((1, H)), full((1, H)), full((1, H)),
                  blk(H), full((H, C)), full((H, C))],
        out_specs=[blk(H), blk(C), blk(C)],
        out_shape=[jax.ShapeDtypeStruct((N, H), f32),
                   jax.ShapeDtypeStruct((N, C), f32),
                   jax.ShapeDtypeStruct((N, C), f32)],
        compiler_params=_SEQ,
    )(y1, ysum1, ysq1, bn_g[0].reshape(1, H), bn_b[0].reshape(1, H),
      h0, wi[1], wj[1])

    s1b, s2b, mnb, mxb = _run_stats(src3, dst3, attr3, b2, te2, with_deg=False)
    y2, ysum2, ysq2 = _run_node(h1, a2, s1b, s2b, mnb, mxb, deg,
                                px[1], pabc[1][0], pabc[1][1], pabc[1][2],
                                postb[1], lin_W[1], lin_b[1].reshape(1, H))

    _, loss2d = pl.pallas_call(
        _head_kernel,
        grid=(NBLK,),
        in_specs=[blk(H), full((1, H)), full((1, H)), full((1, H)), full((1, H)),
                  blk(H), pl.BlockSpec((BN, 1), lambda i: (i, 0)),
                  full((G, 1)), full((H, H)), full((1, H)), full((H, 1)),
                  full((1, 1))],
        out_specs=[full((G, H)), full((1, 1))],
        out_shape=[jax.ShapeDtypeStruct((G, H), f32),
                   jax.ShapeDtypeStruct((1, 1), f32)],
        compiler_params=_SEQ,
    )(y2, ysum2, ysq2, bn_g[1].reshape(1, H), bn_b[1].reshape(1, H),
      h1, batch2, yt, lin1_W, lin1_b.reshape(1, H), lin3_W,
      lin3_b.reshape(1, 1))

    loss = loss2d.reshape(())
    return (loss, loss)
